# baseline (speedup 1.0000x reference)
import jax
import jax.numpy as jnp
from jax import lax
from jax.experimental import pallas as pl
from jax.experimental.pallas import tpu as pltpu
from jax.experimental.pallas import tpu_sc as plsc

_NUM_USERS = 25000
_NUM_ITEMS = 25000
_N = _NUM_USERS + _NUM_ITEMS
_E = 800000
_D = 64
_DH = _D // 2
_ND = _DH // 16
_LANES = 128
_CR = 10
_CHUNK = _CR * _LANES
_NCHUNKS = _E // _CHUNK
_CPT = -(-_NCHUNKS // 16)
_ACC_ROWS = 50176
_ZB = 16
_ZPT = _ACC_ROWS // 16 // _ZB
_WB = 8
_NWB = _N // _WB
_WPT = -(-_NWB // 16)


def _prop_body(idx_ref, vals_ref, e0_ref, o1_ref, o2_ref, o3_ref,
               acc, rowb, colb, valsb, rowsb, zbuf,
               gsem0, gsem1, gsem2, ssem, zsem, wsem):
    c = lax.axis_index("c")
    s = lax.axis_index("s")
    gs = [gsem0, gsem1, gsem2]
    cn = jnp.full((16,), _N, jnp.int32) * c

    def zrow(r, _):
        for d in range(_ND):
            zbuf[r, pl.ds(d * 16, 16)] = jnp.zeros((16,), jnp.float32)
        return _
    lax.fori_loop(0, _ZB, zrow, 0)

    def zero_acc():
        zlast = s * _ZPT + _ZPT - 1
        def zgroup(g, _):
            cps = []
            for t in range(8):
                zc = jnp.minimum(s * _ZPT + g * 8 + t, zlast)
                cps.append(pltpu.async_copy(zbuf,
                                            acc.at[pl.ds(zc * _ZB, _ZB)],
                                            zsem))
            for cp in cps:
                cp.wait()
            return _
        lax.fori_loop(0, -(-_ZPT // 8), zgroup, 0)

    def scale(buf, j):
        def sbody(k, _):
            vv = valsb[pl.ds(j * _LANES + k * 16, 16)]
            for i0 in range(0, 16, 4):
                vs = [vv[i0 + t] for t in range(4)]
                loads = [rowsb[buf, k * 16 + i0 + t, pl.ds(d * 16, 16)]
                         for t in range(4) for d in range(_ND)]
                prods = [loads[t * _ND + d] * vs[t]
                         for t in range(4) for d in range(_ND)]
                for t in range(4):
                    for d in range(_ND):
                        rowsb[buf, k * 16 + i0 + t, pl.ds(d * 16, 16)] = (
                            prods[t * _ND + d])
            return _
        lax.fori_loop(0, _LANES // 16, sbody, 0)

    def stream(src_ref):
        def chunk_body(ci, _):
            e0 = ci * _CHUNK
            pltpu.sync_copy(idx_ref.at[0, pl.ds(e0, _CHUNK)], rowb)
            pltpu.sync_copy(idx_ref.at[1, pl.ds(e0, _CHUNK)], colb)
            pltpu.sync_copy(vals_ref.at[pl.ds(e0, _CHUNK)], valsb)
            def mbody(k, _):
                colb[pl.ds(k * 16, 16)] = colb[pl.ds(k * 16, 16)] + cn
                return _
            lax.fori_loop(0, _CHUNK // 16, mbody, 0)
            cps = {}
            for b in range(2):
                cps[b] = pltpu.async_copy(
                    src_ref.at[colb.at[pl.ds(b * _LANES, _LANES)]],
                    rowsb.at[b], gs[b])
            sc_prev = None
            for j in range(_CR):
                b = j % 3
                cps[j].wait()
                scale(b, j)
                if sc_prev is not None:
                    sc_prev.wait()
                sc_prev = pltpu.async_copy(
                    rowsb.at[b],
                    acc.at[rowb.at[pl.ds(j * _LANES, _LANES)]],
                    ssem, add=True)
                if j + 2 < _CR:
                    cps[j + 2] = pltpu.async_copy(
                        src_ref.at[colb.at[pl.ds((j + 2) * _LANES, _LANES)]],
                        rowsb.at[(j + 2) % 3], gs[(j + 2) % 3])
            sc_prev.wait()
            return _
        lo = s * _CPT
        hi = jnp.minimum(lo + _CPT, _NCHUNKS)
        lax.fori_loop(lo, hi, chunk_body, 0)

    def writeback(dst_ref):
        wlo = s * _WPT
        wlast = jnp.minimum(wlo + _WPT, _NWB) - 1
        def wgroup(g, _):
            cps = []
            for t in range(8):
                wc = jnp.minimum(wlo + g * 8 + t, wlast)
                cps.append(pltpu.async_copy(
                    acc.at[pl.ds(wc * _WB, _WB)],
                    dst_ref.at[pl.ds(c * _N + wc * _WB, _WB)], wsem))
            for cp in cps:
                cp.wait()
            return _
        lax.fori_loop(0, -(-_WPT // 8), wgroup, 0)

    for src_ref, dst_ref in ((e0_ref, o1_ref), (o1_ref, o2_ref),
                             (o2_ref, o3_ref)):
        zero_acc()
        plsc.subcore_barrier()
        stream(src_ref)
        plsc.subcore_barrier()
        writeback(dst_ref)
        plsc.subcore_barrier()


_prop = pl.kernel(
    _prop_body,
    out_type=(jax.ShapeDtypeStruct((2 * _N, _DH), jnp.float32),) * 3,
    mesh=plsc.VectorSubcoreMesh(core_axis_name="c", subcore_axis_name="s"),
    compiler_params=pltpu.CompilerParams(use_tc_tiling_on_sc=False),
    scratch_types=[
        pltpu.VMEM_SHARED((_ACC_ROWS, _DH), jnp.float32),
        pltpu.VMEM((_CHUNK,), jnp.int32),
        pltpu.VMEM((_CHUNK,), jnp.int32),
        pltpu.VMEM((_CHUNK,), jnp.float32),
        pltpu.VMEM((3, _LANES, _DH), jnp.float32),
        pltpu.VMEM((_ZB, _DH), jnp.float32),
    ] + [pltpu.SemaphoreType.DMA] * 6,
)


def _mean_body(a_ref, b_ref, c_ref, d_ref, o_ref):
    o_ref[...] = (a_ref[...] + b_ref[...] + c_ref[...] + d_ref[...]) * 0.25


_mean = pl.pallas_call(
    _mean_body,
    grid=(50,),
    in_specs=[pl.BlockSpec((2 * _N // 50, _DH), lambda i: (i, 0))] * 4,
    out_specs=pl.BlockSpec((2 * _N // 50, _DH), lambda i: (i, 0)),
    out_shape=jax.ShapeDtypeStruct((2 * _N, _DH), jnp.float32),
)


def kernel(adj_indices, adj_values, user_emb, item_emb):
    emb0 = jnp.concatenate(
        [user_emb[:, :_DH], item_emb[:, :_DH],
         user_emb[:, _DH:], item_emb[:, _DH:]], axis=0)
    emb1, emb2, emb3 = _prop(adj_indices, adj_values, emb0)
    mean = _mean(emb0, emb1, emb2, emb3)
    final = jnp.concatenate([mean[:_N], mean[_N:]], axis=1)
    return final[:_NUM_USERS], final[_NUM_USERS:]


# speedup vs baseline: 1.1601x; 1.1601x over previous
import jax
import jax.numpy as jnp
from jax import lax
from jax.experimental import pallas as pl
from jax.experimental.pallas import tpu as pltpu
from jax.experimental.pallas import tpu_sc as plsc

_NUM_USERS = 25000
_NUM_ITEMS = 25000
_N = _NUM_USERS + _NUM_ITEMS
_E = 800000
_D = 64
_DH = _D // 2
_ND = _DH // 16
_LANES = 128
_CR = 10
_CHUNK = _CR * _LANES
_NCHUNKS = _E // _CHUNK
_CPT = -(-_NCHUNKS // 16)
_ACC_ROWS = 50176
_ZB = 16
_ZPT = _ACC_ROWS // 16 // _ZB
_WB = 8
_NWB = _N // _WB
_WPT = -(-_NWB // 16)


def _layer_body(idx_ref, vals_ref, emb_ref, out_ref,
                acc, rowb, colb, valsb, rowsb, zbuf,
                gsem0, gsem1, gsem2, ssem, zsem, wsem):
    c = lax.axis_index("c")
    s = lax.axis_index("s")
    gs = [gsem0, gsem1, gsem2]

    def zrow(r, _):
        for d in range(_ND):
            zbuf[r, pl.ds(d * 16, 16)] = jnp.zeros((16,), jnp.float32)
        return _
    lax.fori_loop(0, _ZB, zrow, 0)

    zlast = s * _ZPT + _ZPT - 1
    def zgroup(g, _):
        cps = []
        for t in range(8):
            zc = jnp.minimum(s * _ZPT + g * 8 + t, zlast)
            cps.append(pltpu.async_copy(zbuf, acc.at[pl.ds(zc * _ZB, _ZB)],
                                        zsem))
        for cp in cps:
            cp.wait()
        return _
    lax.fori_loop(0, -(-_ZPT // 8), zgroup, 0)
    plsc.subcore_barrier()

    def scale(buf, j):
        def sbody(k, _):
            vv = valsb[pl.ds(j * _LANES + k * 16, 16)]
            for i0 in range(0, 16, 4):
                vs = [vv[i0 + t] for t in range(4)]
                loads = [rowsb[buf, k * 16 + i0 + t, pl.ds(d * 16, 16)]
                         for t in range(4) for d in range(_ND)]
                prods = [loads[t * _ND + d] * vs[t]
                         for t in range(4) for d in range(_ND)]
                for t in range(4):
                    for d in range(_ND):
                        rowsb[buf, k * 16 + i0 + t, pl.ds(d * 16, 16)] = (
                            prods[t * _ND + d])
            return _
        lax.fori_loop(0, _LANES // 16, sbody, 0)

    def chunk_body(ci, _):
        e0 = ci * _CHUNK
        pltpu.sync_copy(idx_ref.at[0, pl.ds(e0, _CHUNK)], rowb)
        pltpu.sync_copy(idx_ref.at[1, pl.ds(e0, _CHUNK)], colb)
        pltpu.sync_copy(vals_ref.at[pl.ds(e0, _CHUNK)], valsb)
        def mbody(k, _):
            v = colb[pl.ds(k * 16, 16)]
            colb[pl.ds(k * 16, 16)] = v + v + c
            return _
        lax.fori_loop(0, _CHUNK // 16, mbody, 0)
        cps = {}
        for b in range(2):
            cps[b] = pltpu.async_copy(
                emb_ref.at[colb.at[pl.ds(b * _LANES, _LANES)]],
                rowsb.at[b], gs[b])
        sc_prev = None
        for j in range(_CR):
            b = j % 3
            cps[j].wait()
            scale(b, j)
            if sc_prev is not None:
                sc_prev.wait()
            sc_prev = pltpu.async_copy(
                rowsb.at[b], acc.at[rowb.at[pl.ds(j * _LANES, _LANES)]],
                ssem, add=True)
            if j + 2 < _CR:
                cps[j + 2] = pltpu.async_copy(
                    emb_ref.at[colb.at[pl.ds((j + 2) * _LANES, _LANES)]],
                    rowsb.at[(j + 2) % 3], gs[(j + 2) % 3])
        sc_prev.wait()
        return _

    lo = s * _CPT
    hi = jnp.minimum(lo + _CPT, _NCHUNKS)
    lax.fori_loop(lo, hi, chunk_body, 0)
    plsc.subcore_barrier()

    wlo = s * _WPT
    wlast = jnp.minimum(wlo + _WPT, _NWB) - 1
    def wgroup(g, _):
        cps = []
        for t in range(8):
            wc = jnp.minimum(wlo + g * 8 + t, wlast)
            cps.append(pltpu.async_copy(
                acc.at[pl.ds(wc * _WB, _WB)],
                out_ref.at[pl.ds(wc * _WB, _WB), pl.ds(c * _DH, _DH)],
                wsem))
        for cp in cps:
            cp.wait()
        return _
    lax.fori_loop(0, -(-_WPT // 8), wgroup, 0)


_layer = pl.kernel(
    _layer_body,
    out_type=jax.ShapeDtypeStruct((_N, _D), jnp.float32),
    mesh=plsc.VectorSubcoreMesh(core_axis_name="c", subcore_axis_name="s"),
    compiler_params=pltpu.CompilerParams(use_tc_tiling_on_sc=False),
    scratch_types=[
        pltpu.VMEM_SHARED((_ACC_ROWS, _DH), jnp.float32),
        pltpu.VMEM((_CHUNK,), jnp.int32),
        pltpu.VMEM((_CHUNK,), jnp.int32),
        pltpu.VMEM((_CHUNK,), jnp.float32),
        pltpu.VMEM((3, _LANES, _DH), jnp.float32),
        pltpu.VMEM((_ZB, _DH), jnp.float32),
    ] + [pltpu.SemaphoreType.DMA] * 6,
)


def _mean_body(a_ref, b_ref, c_ref, d_ref, o_ref):
    o_ref[...] = (a_ref[...] + b_ref[...] + c_ref[...] + d_ref[...]) * 0.25


_mean = pl.pallas_call(
    _mean_body,
    grid=(50,),
    in_specs=[pl.BlockSpec((_N // 50, _D), lambda i: (i, 0))] * 4,
    out_specs=pl.BlockSpec((_N // 50, _D), lambda i: (i, 0)),
    out_shape=jax.ShapeDtypeStruct((_N, _D), jnp.float32),
)


def kernel(adj_indices, adj_values, user_emb, item_emb):
    emb0 = jnp.concatenate([user_emb, item_emb], axis=0)
    emb1 = _layer(adj_indices, adj_values, emb0.reshape(2 * _N, _DH))
    emb2 = _layer(adj_indices, adj_values, emb1.reshape(2 * _N, _DH))
    emb3 = _layer(adj_indices, adj_values, emb2.reshape(2 * _N, _DH))
    final = _mean(emb0, emb1, emb2, emb3)
    return final[:_NUM_USERS], final[_NUM_USERS:]


# speedup vs baseline: 1.3830x; 1.1921x over previous
import jax
import jax.numpy as jnp
from jax import lax
from jax.experimental import pallas as pl
from jax.experimental.pallas import tpu as pltpu
from jax.experimental.pallas import tpu_sc as plsc

_NUM_USERS = 25000
_NUM_ITEMS = 25000
_N = _NUM_USERS + _NUM_ITEMS
_E = 800000
_D = 64
_DH = _D // 2
_ND = _DH // 16
_LANES = 128
_CR = 10
_CHUNK = _CR * _LANES
_NCHUNKS = _E // _CHUNK
_CPT = -(-_NCHUNKS // 16)
_ACC_ROWS = 50176
_ZB = 16
_ZPT = _ACC_ROWS // 16 // _ZB
_WB = 8
_NWB = _N // _WB
_WPT = -(-_NWB // 16)


def _layer_body(idx_ref, vals_ref, emb_ref, out_ref,
                acc, rowb, colb, valsb, rowsb, zbuf,
                gsem0, gsem1, gsem2, ssem, zsem, wsem, dsem):
    c = lax.axis_index("c")
    s = lax.axis_index("s")
    gs = [gsem0, gsem1, gsem2]

    def zrow(r, _):
        for d in range(_ND):
            zbuf[r, pl.ds(d * 16, 16)] = jnp.zeros((16,), jnp.float32)
        return _
    lax.fori_loop(0, _ZB, zrow, 0)

    zlast = s * _ZPT + _ZPT - 1
    def zgroup(g, _):
        cps = []
        for t in range(8):
            zc = jnp.minimum(s * _ZPT + g * 8 + t, zlast)
            cps.append(pltpu.async_copy(zbuf, acc.at[pl.ds(zc * _ZB, _ZB)],
                                        zsem))
        for cp in cps:
            cp.wait()
        return _
    lax.fori_loop(0, -(-_ZPT // 8), zgroup, 0)
    plsc.subcore_barrier()

    def scale(buf, j, ob):
        def sbody(k, _):
            vv = valsb[pl.ds(ob + j * _LANES + k * 16, 16)]
            for i0 in range(0, 16, 4):
                vs = [vv[i0 + t] for t in range(4)]
                loads = [rowsb[buf, k * 16 + i0 + t, pl.ds(d * 16, 16)]
                         for t in range(4) for d in range(_ND)]
                prods = [loads[t * _ND + d] * vs[t]
                         for t in range(4) for d in range(_ND)]
                for t in range(4):
                    for d in range(_ND):
                        rowsb[buf, k * 16 + i0 + t, pl.ds(d * 16, 16)] = (
                            prods[t * _ND + d])
            return _
        lax.fori_loop(0, _LANES // 16, sbody, 0)

    lo = s * _CPT
    hi = jnp.minimum(lo + _CPT, _NCHUNKS)

    def prefetch(ci):
        ob = jnp.bitwise_and(ci, 1) * _CHUNK
        e0 = jnp.minimum(ci, hi - 1) * _CHUNK
        pltpu.async_copy(idx_ref.at[0, pl.ds(e0, _CHUNK)],
                         rowb.at[pl.ds(ob, _CHUNK)], dsem)
        pltpu.async_copy(idx_ref.at[1, pl.ds(e0, _CHUNK)],
                         colb.at[pl.ds(ob, _CHUNK)], dsem)
        pltpu.async_copy(vals_ref.at[pl.ds(e0, _CHUNK)],
                         valsb.at[pl.ds(ob, _CHUNK)], dsem)

    def drain(n):
        def dbody(i, _):
            pltpu.make_async_copy(vals_ref.at[pl.ds(0, _CHUNK)],
                                  valsb.at[pl.ds(0, _CHUNK)], dsem).wait()
            return _
        lax.fori_loop(0, n, dbody, 0)

    def chunk_body(ci, _):
        ob = jnp.bitwise_and(ci, 1) * _CHUNK
        drain(3)
        prefetch(ci + 1)
        def mbody(k, _):
            v = colb[pl.ds(ob + k * 16, 16)]
            colb[pl.ds(ob + k * 16, 16)] = v + v + c
            return _
        lax.fori_loop(0, _CHUNK // 16, mbody, 0)
        cps = {}
        for b in range(2):
            cps[b] = pltpu.async_copy(
                emb_ref.at[colb.at[pl.ds(ob + b * _LANES, _LANES)]],
                rowsb.at[b], gs[b])
        sc_prev = None
        for j in range(_CR):
            b = j % 3
            cps[j].wait()
            scale(b, j, ob)
            if sc_prev is not None:
                sc_prev.wait()
            sc_prev = pltpu.async_copy(
                rowsb.at[b],
                acc.at[rowb.at[pl.ds(ob + j * _LANES, _LANES)]],
                ssem, add=True)
            if j + 2 < _CR:
                cps[j + 2] = pltpu.async_copy(
                    emb_ref.at[colb.at[pl.ds(ob + (j + 2) * _LANES, _LANES)]],
                    rowsb.at[(j + 2) % 3], gs[(j + 2) % 3])
        sc_prev.wait()
        return _

    prefetch(lo)
    lax.fori_loop(lo, hi, chunk_body, 0)
    drain(3)
    plsc.subcore_barrier()

    wlo = s * _WPT
    wlast = jnp.minimum(wlo + _WPT, _NWB) - 1
    def wgroup(g, _):
        cps = []
        for t in range(8):
            wc = jnp.minimum(wlo + g * 8 + t, wlast)
            cps.append(pltpu.async_copy(
                acc.at[pl.ds(wc * _WB, _WB)],
                out_ref.at[pl.ds(wc * _WB, _WB), pl.ds(c * _DH, _DH)],
                wsem))
        for cp in cps:
            cp.wait()
        return _
    lax.fori_loop(0, -(-_WPT // 8), wgroup, 0)


_layer = pl.kernel(
    _layer_body,
    out_type=jax.ShapeDtypeStruct((_N, _D), jnp.float32),
    mesh=plsc.VectorSubcoreMesh(core_axis_name="c", subcore_axis_name="s"),
    compiler_params=pltpu.CompilerParams(use_tc_tiling_on_sc=False),
    scratch_types=[
        pltpu.VMEM_SHARED((_ACC_ROWS, _DH), jnp.float32),
        pltpu.VMEM((2 * _CHUNK,), jnp.int32),
        pltpu.VMEM((2 * _CHUNK,), jnp.int32),
        pltpu.VMEM((2 * _CHUNK,), jnp.float32),
        pltpu.VMEM((3, _LANES, _DH), jnp.float32),
        pltpu.VMEM((_ZB, _DH), jnp.float32),
    ] + [pltpu.SemaphoreType.DMA] * 7,
)


def _mean_body(a_ref, b_ref, c_ref, d_ref, o_ref):
    o_ref[...] = (a_ref[...] + b_ref[...] + c_ref[...] + d_ref[...]) * 0.25


_mean = pl.pallas_call(
    _mean_body,
    grid=(50,),
    in_specs=[pl.BlockSpec((_N // 50, _D), lambda i: (i, 0))] * 4,
    out_specs=pl.BlockSpec((_N // 50, _D), lambda i: (i, 0)),
    out_shape=jax.ShapeDtypeStruct((_N, _D), jnp.float32),
)


def kernel(adj_indices, adj_values, user_emb, item_emb):
    emb0 = jnp.concatenate([user_emb, item_emb], axis=0)
    emb1 = _layer(adj_indices, adj_values, emb0.reshape(2 * _N, _DH))
    emb2 = _layer(adj_indices, adj_values, emb1.reshape(2 * _N, _DH))
    emb3 = _layer(adj_indices, adj_values, emb2.reshape(2 * _N, _DH))
    final = _mean(emb0, emb1, emb2, emb3)
    return final[:_NUM_USERS], final[_NUM_USERS:]
